# baseline (speedup 1.0000x reference)
import functools

import jax
import jax.numpy as jnp
from jax import lax
from jax.experimental import pallas as pl
from jax.experimental.pallas import tpu as pltpu
from jax.experimental.pallas import tpu_sc as plsc

D_MODEL = 4096
N_EXP = 64
K_TOP = 8
N_TOK = 8192
CAP = N_TOK // N_EXP
BLK = 256
N_BLK = N_TOK // BLK

_NEG_INF = float("-inf")


def _tree_sum8(vals):
    a = [vals[0] + vals[1], vals[2] + vals[3], vals[4] + vals[5], vals[6] + vals[7]]
    return (a[0] + a[1]) + (a[2] + a[3])


def _stage_a(x_ref, w_ref, idx_ref, prob_ref, keys_ref, colsum_ref, zacc_ref):
    pid = pl.program_id(0)
    x = x_ref[...]
    w = w_ref[...]
    logits = lax.dot_general(
        x, w, (((1,), (1,)), ((), ())), preferred_element_type=jnp.float32
    )

    lane = lax.broadcasted_iota(jnp.int32, (BLK, N_EXP), 1)

    m64 = jnp.max(logits, axis=1, keepdims=True)
    ex = jnp.exp(logits - m64)
    s64 = jnp.sum(ex, axis=1, keepdims=True)
    probs = ex / s64
    col_partial = jnp.sum(probs, axis=0, keepdims=True)
    lse = m64 + jnp.log(s64)
    z_partial = jnp.sum(lse * lse)

    l = logits
    vals = []
    idxs = []
    for _ in range(K_TOP):
        m = jnp.max(l, axis=1, keepdims=True)
        am = jnp.min(jnp.where(l == m, lane, N_EXP), axis=1, keepdims=True)
        vals.append(m)
        idxs.append(am)
        l = jnp.where(lane == am, _NEG_INF, l)

    exs = [jnp.exp(v - vals[0]) for v in vals]
    s8 = _tree_sum8(exs)
    ps = [e / s8 for e in exs]
    t8 = _tree_sum8(ps)
    t8 = jnp.maximum(t8, 1e-8)
    qs = [p / t8 for p in ps]

    keys = jnp.full((BLK, N_EXP), -1, jnp.int32)
    for k in range(K_TOP):
        kb = lax.bitcast_convert_type(qs[k], jnp.int32)
        keys = jnp.where(lane == idxs[k], kb, keys)

    idx_ref[...] = jnp.concatenate(idxs, axis=1)
    prob_ref[...] = jnp.concatenate(qs, axis=1)
    keys_ref[...] = keys

    @pl.when(pid == 0)
    def _():
        colsum_ref[...] = jnp.zeros_like(colsum_ref)
        zacc_ref[...] = jnp.zeros_like(zacc_ref)

    colsum_ref[...] += jnp.broadcast_to(col_partial, colsum_ref.shape)
    zacc_ref[...] += z_partial


_N_VREG = N_TOK // 16


def _sc_filter_body(keysT_ref, out_ref, keys_v, res_v):
    c = lax.axis_index("c")
    s = lax.axis_index("s")
    wid = s * 2 + c
    lane16 = lax.iota(jnp.int32, 16)

    def vec_total(acc):
        t = acc[0]
        for l in range(1, 16):
            t = t + acc[l]
        return t

    res = jnp.zeros((16,), jnp.int32)
    for j in range(2):
        e = wid * 2 + j
        pltpu.sync_copy(keysT_ref.at[e], keys_v)

        def cnt_gt(t):
            def body(i, acc):
                k = keys_v[pl.ds(i * 16, 16)]
                return acc + jnp.where(k > t, 1, 0).astype(jnp.int32)

            acc = lax.fori_loop(0, _N_VREG, body, jnp.zeros((16,), jnp.int32))
            return vec_total(acc)

        def bs_body(_, lohi):
            lo, hi = lohi
            mid = lo + (hi - lo) // 2
            small = cnt_gt(mid) < CAP
            return (jnp.where(small, lo, mid), jnp.where(small, mid, hi))

        lo, hi = lax.fori_loop(
            0, 32, bs_body, (jnp.int32(-2), jnp.int32(1 << 30))
        )
        v = hi
        r = CAP - cnt_gt(v)

        def cnt_tle(t):
            def body(i, acc):
                k = keys_v[pl.ds(i * 16, 16)]
                tokv = lane16 + i * 16
                m = (k == v) & (tokv <= t)
                return acc + jnp.where(m, 1, 0).astype(jnp.int32)

            acc = lax.fori_loop(0, _N_VREG, body, jnp.zeros((16,), jnp.int32))
            return vec_total(acc)

        def ts_body(_, lohi):
            lo, hi = lohi
            mid = lo + (hi - lo) // 2
            ok = cnt_tle(mid) <= r
            return (jnp.where(ok, mid, lo), jnp.where(ok, hi, mid))

        lo2, _ = lax.fori_loop(
            0, 13, ts_body, (jnp.int32(-1), jnp.int32(N_TOK))
        )

        res = jnp.where(lane16 == 2 * j, v, res)
        res = jnp.where(lane16 == 2 * j + 1, lo2, res)
    res_v[...] = res
    pltpu.sync_copy(res_v, out_ref.at[wid])


_sc_filter = functools.partial(
    pl.kernel,
    mesh=plsc.VectorSubcoreMesh(core_axis_name="c", subcore_axis_name="s"),
    out_type=jax.ShapeDtypeStruct((32, 16), jnp.int32),
    scratch_types=[
        pltpu.VMEM((N_TOK,), jnp.int32),
        pltpu.VMEM((16,), jnp.int32),
    ],
)(_sc_filter_body)


def _stage_c(idx_ref, prob_ref, v_ref, t_ref, colsum_ref, zacc_ref,
             mod_idx_ref, mod_prob_ref, tpe_ref, lb_ref, zl_ref):
    lane = lax.broadcasted_iota(jnp.int32, (N_TOK, N_EXP), 1)
    tokcol = lax.broadcasted_iota(jnp.int32, (N_TOK, 1), 0)
    v = v_ref[0:1, :]
    tstar = t_ref[0:1, :]

    mod_idx_cols = []
    mod_prob_cols = []
    keep0 = None
    sel0 = None
    for k in range(K_TOP):
        e_k = idx_ref[:, k : k + 1]
        p_k = prob_ref[:, k : k + 1]
        sel = lane == e_k
        v_k = jnp.sum(jnp.where(sel, v, 0), axis=1, keepdims=True)
        t_k = jnp.sum(jnp.where(sel, tstar, 0), axis=1, keepdims=True)
        key_k = lax.bitcast_convert_type(p_k, jnp.int32)
        keep = (key_k > v_k) | ((key_k == v_k) & (tokcol <= t_k))
        if k == 0:
            keep0 = keep
            sel0 = sel
        mod_prob_cols.append(jnp.where(keep, p_k, 0.0))
        mod_idx_cols.append(jnp.where(keep, e_k, -1))

    mod_idx_ref[...] = jnp.concatenate(mod_idx_cols, axis=1)
    mod_prob_ref[...] = jnp.concatenate(mod_prob_cols, axis=1)

    tpe = jnp.sum(
        jnp.where(sel0 & keep0, 1.0, 0.0).astype(jnp.float32), axis=0, keepdims=True
    )
    tpe_ref[...] = jnp.broadcast_to(tpe, tpe_ref.shape)

    colsum = colsum_ref[0:1, :]
    lb = jnp.sum(colsum * tpe) * (0.01 / (N_TOK * N_EXP))
    lb_ref[...] = jnp.broadcast_to(lb, lb_ref.shape)
    zl = (zacc_ref[0, 0] / N_TOK) * 0.001
    zl_ref[...] = jnp.broadcast_to(zl, zl_ref.shape)


def _stage_b(idx_ref, prob_ref, keys_ref, colsum_ref, zacc_ref,
             mod_idx_ref, mod_prob_ref, tpe_ref, lb_ref, zl_ref):
    kmat = keys_ref[...]
    lane = lax.broadcasted_iota(jnp.int32, (N_TOK, N_EXP), 1)
    tok = lax.broadcasted_iota(jnp.int32, (N_TOK, N_EXP), 0)
    tokcol = lax.broadcasted_iota(jnp.int32, (N_TOK, 1), 0)

    def cnt_gt(t):
        return jnp.sum((kmat > t).astype(jnp.int32), axis=0, keepdims=True)

    lo0 = jnp.full((1, N_EXP), -2, jnp.int32)
    hi0 = jnp.full((1, N_EXP), 1 << 30, jnp.int32)

    def bs_body(_, carry):
        lo, hi = carry
        mid = lo + (hi - lo) // 2
        small = cnt_gt(mid) < CAP
        return jnp.where(small, lo, mid), jnp.where(small, mid, hi)

    lo, hi = lax.fori_loop(0, 32, bs_body, (lo0, hi0))
    v = hi
    r = CAP - cnt_gt(v)

    tie = kmat == v

    def cnt_le(t):
        return jnp.sum((tie & (tok <= t)).astype(jnp.int32), axis=0, keepdims=True)

    lo0t = jnp.full((1, N_EXP), -1, jnp.int32)
    hi0t = jnp.full((1, N_EXP), N_TOK, jnp.int32)

    def ts_body(_, carry):
        lo, hi = carry
        mid = lo + (hi - lo) // 2
        ok = cnt_le(mid) <= r
        return jnp.where(ok, mid, lo), jnp.where(ok, hi, mid)

    lo, hi = lax.fori_loop(0, 13, ts_body, (lo0t, hi0t))
    tstar = lo

    mod_idx_cols = []
    mod_prob_cols = []
    keep0 = None
    sel0 = None
    for k in range(K_TOP):
        e_k = idx_ref[:, k : k + 1]
        p_k = prob_ref[:, k : k + 1]
        sel = lane == e_k
        v_k = jnp.sum(jnp.where(sel, v, 0), axis=1, keepdims=True)
        t_k = jnp.sum(jnp.where(sel, tstar, 0), axis=1, keepdims=True)
        key_k = lax.bitcast_convert_type(p_k, jnp.int32)
        keep = (key_k > v_k) | ((key_k == v_k) & (tokcol <= t_k))
        if k == 0:
            keep0 = keep
            sel0 = sel
        mod_prob_cols.append(jnp.where(keep, p_k, 0.0))
        mod_idx_cols.append(jnp.where(keep, e_k, -1))

    mod_idx_ref[...] = jnp.concatenate(mod_idx_cols, axis=1)
    mod_prob_ref[...] = jnp.concatenate(mod_prob_cols, axis=1)

    tpe = jnp.sum(
        jnp.where(sel0 & keep0, 1.0, 0.0).astype(jnp.float32), axis=0, keepdims=True
    )
    tpe_ref[...] = jnp.broadcast_to(tpe, tpe_ref.shape)

    colsum = colsum_ref[0:1, :]
    lb = jnp.sum(colsum * tpe) * (0.01 / (N_TOK * N_EXP))
    lb_ref[...] = jnp.broadcast_to(lb, lb_ref.shape)
    zl = (zacc_ref[0, 0] / N_TOK) * 0.001
    zl_ref[...] = jnp.broadcast_to(zl, zl_ref.shape)


@jax.jit
def kernel(x, W):
    idx, prob, keys, colsum, zacc = pl.pallas_call(
        _stage_a,
        grid=(N_BLK,),
        in_specs=[
            pl.BlockSpec((BLK, D_MODEL), lambda i: (i, 0)),
            pl.BlockSpec((N_EXP, D_MODEL), lambda i: (0, 0)),
        ],
        out_specs=[
            pl.BlockSpec((BLK, K_TOP), lambda i: (i, 0)),
            pl.BlockSpec((BLK, K_TOP), lambda i: (i, 0)),
            pl.BlockSpec((BLK, N_EXP), lambda i: (i, 0)),
            pl.BlockSpec((8, N_EXP), lambda i: (0, 0)),
            pl.BlockSpec((8, 128), lambda i: (0, 0)),
        ],
        out_shape=[
            jax.ShapeDtypeStruct((N_TOK, K_TOP), jnp.int32),
            jax.ShapeDtypeStruct((N_TOK, K_TOP), jnp.float32),
            jax.ShapeDtypeStruct((N_TOK, N_EXP), jnp.int32),
            jax.ShapeDtypeStruct((8, N_EXP), jnp.float32),
            jax.ShapeDtypeStruct((8, 128), jnp.float32),
        ],
    )(x, W)

    keysT = jnp.transpose(keys)
    vt = _sc_filter(keysT)
    v64 = jnp.broadcast_to(vt[:, 0:4:2].reshape(1, N_EXP), (8, N_EXP))
    t64 = jnp.broadcast_to(vt[:, 1:4:2].reshape(1, N_EXP), (8, N_EXP))

    mod_idx, mod_prob, tpe, lb, zl = pl.pallas_call(
        _stage_c,
        out_shape=[
            jax.ShapeDtypeStruct((N_TOK, K_TOP), jnp.int32),
            jax.ShapeDtypeStruct((N_TOK, K_TOP), jnp.float32),
            jax.ShapeDtypeStruct((8, N_EXP), jnp.float32),
            jax.ShapeDtypeStruct((8, 128), jnp.float32),
            jax.ShapeDtypeStruct((8, 128), jnp.float32),
        ],
    )(idx, prob, v64, t64, colsum, zacc)

    return (
        mod_idx,
        mod_prob,
        lb[0, 0],
        zl[0, 0],
        tpe[0, :],
    )


# speedup vs baseline: 1.6483x; 1.6483x over previous
import functools

import jax
import jax.numpy as jnp
from jax import lax
from jax.experimental import pallas as pl
from jax.experimental.pallas import tpu as pltpu
from jax.experimental.pallas import tpu_sc as plsc

D_MODEL = 4096
N_EXP = 64
K_TOP = 8
N_TOK = 8192
CAP = N_TOK // N_EXP
BLK = 256
N_BLK = N_TOK // BLK

_NEG_INF = float("-inf")


def _tree_sum8(vals):
    a = [vals[0] + vals[1], vals[2] + vals[3], vals[4] + vals[5], vals[6] + vals[7]]
    return (a[0] + a[1]) + (a[2] + a[3])


def _stage_a(x_ref, w_ref, idx_ref, prob_ref, keys_ref, colsum_ref, zacc_ref):
    pid = pl.program_id(0)
    x = x_ref[...]
    w = w_ref[...]
    logits = lax.dot_general(
        x, w, (((1,), (1,)), ((), ())), preferred_element_type=jnp.float32
    )

    lane = lax.broadcasted_iota(jnp.int32, (BLK, N_EXP), 1)

    m64 = jnp.max(logits, axis=1, keepdims=True)
    ex = jnp.exp(logits - m64)
    s64 = jnp.sum(ex, axis=1, keepdims=True)
    probs = ex / s64
    col_partial = jnp.sum(probs, axis=0, keepdims=True)
    lse = m64 + jnp.log(s64)
    z_partial = jnp.sum(lse * lse)

    l = logits
    vals = []
    idxs = []
    for _ in range(K_TOP):
        m = jnp.max(l, axis=1, keepdims=True)
        am = jnp.min(jnp.where(l == m, lane, N_EXP), axis=1, keepdims=True)
        vals.append(m)
        idxs.append(am)
        l = jnp.where(lane == am, _NEG_INF, l)

    exs = [jnp.exp(v - vals[0]) for v in vals]
    s8 = _tree_sum8(exs)
    ps = [e / s8 for e in exs]
    t8 = _tree_sum8(ps)
    t8 = jnp.maximum(t8, 1e-8)
    qs = [p / t8 for p in ps]

    keys = jnp.full((BLK, N_EXP), -1, jnp.int32)
    for k in range(K_TOP):
        kb = lax.bitcast_convert_type(qs[k], jnp.int32)
        keys = jnp.where(lane == idxs[k], kb, keys)

    idx_ref[...] = jnp.concatenate(idxs, axis=1)
    prob_ref[...] = jnp.concatenate(qs, axis=1)
    keys_ref[...] = keys

    @pl.when(pid == 0)
    def _():
        colsum_ref[...] = jnp.zeros_like(colsum_ref)
        zacc_ref[...] = jnp.zeros_like(zacc_ref)

    colsum_ref[...] += jnp.broadcast_to(col_partial, colsum_ref.shape)
    zacc_ref[...] += z_partial


_N_VREG = N_TOK // 16


_BSTRIDE = 144


def _sc_filter_body(keysT_ref, out_ref, keys_v, cand_k, cand_t, hist, res_v):
    c = lax.axis_index("c")
    s = lax.axis_index("s")
    wid = s * 2 + c
    lane16 = lax.iota(jnp.int32, 16)
    ones16 = jnp.ones((16,), jnp.int32)
    zeros16 = jnp.zeros((16,), jnp.int32)
    true16 = jnp.ones((16,), jnp.bool_)

    def vec_total(acc):
        t = acc[0]
        for l in range(1, 16):
            t = t + acc[l]
        return t

    res = jnp.zeros((16,), jnp.int32)
    for j in range(2):
        e = wid * 2 + j
        pltpu.sync_copy(keysT_ref.at[e], keys_v)

        def zbody(i, carry):
            hist[pl.ds(i * 16, 16)] = zeros16
            return carry

        lax.fori_loop(0, _BSTRIDE, zbody, 0, unroll=4)

        def hbody(i, carry):
            k = keys_v[pl.ds(i * 16, 16)]
            b = lax.shift_right_arithmetic(k, 23) + 1
            idx = lane16 * _BSTRIDE + b
            h = plsc.load_gather(hist, [idx])
            plsc.store_scatter(hist, [idx], h + 1)
            return carry

        lax.fori_loop(0, _N_VREG, hbody, 0, unroll=4)

        tot = [zeros16] * 9
        for l in range(16):
            for bv in range(9):
                tot[bv] = tot[bv] + hist[pl.ds(l * _BSTRIDE + bv * 16, 16)]

        tots = [vec_total(t) for t in tot]
        suffix = [None] * 9
        acc_s = jnp.int32(0)
        for v_i in range(8, -1, -1):
            suffix[v_i] = acc_s
            acc_s = acc_s + tots[v_i]
        vstar = jnp.int32(0)
        for v_i in range(9):
            cond = (suffix[v_i] < CAP) & (suffix[v_i] + tots[v_i] >= CAP)
            vstar = jnp.where(cond, v_i, vstar)
        tv = zeros16
        sfx = jnp.int32(0)
        for v_i in range(9):
            m_v = vstar == v_i
            tv = jnp.where(m_v, tot[v_i], tv)
            sfx = jnp.where(m_v, suffix[v_i], sfx)
        bstar = jnp.int32(0)
        c_above = jnp.int32(0)
        cum = sfx
        for l in range(15, -1, -1):
            tl = tv[l]
            cond = (cum < CAP) & (cum + tl >= CAP)
            bstar = jnp.where(cond, vstar * 16 + l, bstar)
            c_above = jnp.where(cond, cum, c_above)
            cum = cum + tl

        def cbody(i, cnt):
            k = keys_v[pl.ds(i * 16, 16)]
            b = lax.shift_right_arithmetic(k, 23) + 1
            m = b == bstar
            plsc.store_compressed(cand_k.at[pl.ds(cnt, 16)], k, mask=m)
            plsc.store_compressed(cand_t.at[pl.ds(cnt, 16)], lane16 + i * 16, mask=m)
            return cnt + plsc.all_reduce_population_count(m)[0]

        m_cnt = lax.fori_loop(0, _N_VREG, cbody, jnp.int32(0), unroll=2)
        nv_c = (m_cnt + 15) // 16

        lo0 = jnp.where(bstar == 0, jnp.int32(-2), ((bstar - 1) << 23) - 1)
        hi0 = jnp.where(bstar == 0, jnp.int32(0), bstar << 23)
        cap_r = CAP - c_above

        def cnt_gt(t):
            def body(i, acc):
                k = cand_k[pl.ds(i * 16, 16)]
                g = (lane16 + i * 16) < m_cnt
                return acc + jnp.where((k > t) & g, 1, 0).astype(jnp.int32)

            acc = lax.fori_loop(0, nv_c, body, zeros16)
            return vec_total(acc)

        def bs_body(_, lohi):
            lo, hi = lohi
            mid = lo + (hi - lo) // 2
            small = cnt_gt(mid) < cap_r
            return (jnp.where(small, lo, mid), jnp.where(small, mid, hi))

        lo, hi = lax.fori_loop(0, 24, bs_body, (lo0, hi0))
        v = hi
        r = cap_r - cnt_gt(v)

        def cnt_tle(t):
            def body(i, acc):
                k = cand_k[pl.ds(i * 16, 16)]
                ct = cand_t[pl.ds(i * 16, 16)]
                g = (lane16 + i * 16) < m_cnt
                m = (k == v) & (ct <= t) & g
                return acc + jnp.where(m, 1, 0).astype(jnp.int32)

            acc = lax.fori_loop(0, nv_c, body, zeros16)
            return vec_total(acc)

        def ts_body(_, lohi):
            lo, hi = lohi
            mid = lo + (hi - lo) // 2
            ok = cnt_tle(mid) <= r
            return (jnp.where(ok, mid, lo), jnp.where(ok, hi, mid))

        lo2, _ = lax.fori_loop(
            0, 13, ts_body, (jnp.int32(-1), jnp.int32(N_TOK))
        )

        res = jnp.where(lane16 == 2 * j, v, res)
        res = jnp.where(lane16 == 2 * j + 1, lo2, res)
    res_v[...] = res
    pltpu.sync_copy(res_v, out_ref.at[wid])


_sc_filter = functools.partial(
    pl.kernel,
    mesh=plsc.VectorSubcoreMesh(core_axis_name="c", subcore_axis_name="s"),
    out_type=jax.ShapeDtypeStruct((32, 16), jnp.int32),
    compiler_params=pltpu.CompilerParams(needs_layout_passes=False),
    scratch_types=[
        pltpu.VMEM((N_TOK,), jnp.int32),
        pltpu.VMEM((N_TOK + 16,), jnp.int32),
        pltpu.VMEM((N_TOK + 16,), jnp.int32),
        pltpu.VMEM((16 * _BSTRIDE,), jnp.int32),
        pltpu.VMEM((16,), jnp.int32),
    ],
)(_sc_filter_body)


def _stage_c(idx_ref, prob_ref, v_ref, t_ref, colsum_ref, zacc_ref,
             mod_idx_ref, mod_prob_ref, tpe_ref, lb_ref, zl_ref):
    lane = lax.broadcasted_iota(jnp.int32, (N_TOK, N_EXP), 1)
    tokcol = lax.broadcasted_iota(jnp.int32, (N_TOK, 1), 0)
    v = v_ref[0:1, :]
    tstar = t_ref[0:1, :]

    mod_idx_cols = []
    mod_prob_cols = []
    keep0 = None
    sel0 = None
    for k in range(K_TOP):
        e_k = idx_ref[:, k : k + 1]
        p_k = prob_ref[:, k : k + 1]
        sel = lane == e_k
        v_k = jnp.sum(jnp.where(sel, v, 0), axis=1, keepdims=True)
        t_k = jnp.sum(jnp.where(sel, tstar, 0), axis=1, keepdims=True)
        key_k = lax.bitcast_convert_type(p_k, jnp.int32)
        keep = (key_k > v_k) | ((key_k == v_k) & (tokcol <= t_k))
        if k == 0:
            keep0 = keep
            sel0 = sel
        mod_prob_cols.append(jnp.where(keep, p_k, 0.0))
        mod_idx_cols.append(jnp.where(keep, e_k, -1))

    mod_idx_ref[...] = jnp.concatenate(mod_idx_cols, axis=1)
    mod_prob_ref[...] = jnp.concatenate(mod_prob_cols, axis=1)

    tpe = jnp.sum(
        jnp.where(sel0 & keep0, 1.0, 0.0).astype(jnp.float32), axis=0, keepdims=True
    )
    tpe_ref[...] = jnp.broadcast_to(tpe, tpe_ref.shape)

    colsum = colsum_ref[0:1, :]
    lb = jnp.sum(colsum * tpe) * (0.01 / (N_TOK * N_EXP))
    lb_ref[...] = jnp.broadcast_to(lb, lb_ref.shape)
    zl = (zacc_ref[0, 0] / N_TOK) * 0.001
    zl_ref[...] = jnp.broadcast_to(zl, zl_ref.shape)


def _stage_b(idx_ref, prob_ref, keys_ref, colsum_ref, zacc_ref,
             mod_idx_ref, mod_prob_ref, tpe_ref, lb_ref, zl_ref):
    kmat = keys_ref[...]
    lane = lax.broadcasted_iota(jnp.int32, (N_TOK, N_EXP), 1)
    tok = lax.broadcasted_iota(jnp.int32, (N_TOK, N_EXP), 0)
    tokcol = lax.broadcasted_iota(jnp.int32, (N_TOK, 1), 0)

    def cnt_gt(t):
        return jnp.sum((kmat > t).astype(jnp.int32), axis=0, keepdims=True)

    lo0 = jnp.full((1, N_EXP), -2, jnp.int32)
    hi0 = jnp.full((1, N_EXP), 1 << 30, jnp.int32)

    def bs_body(_, carry):
        lo, hi = carry
        mid = lo + (hi - lo) // 2
        small = cnt_gt(mid) < CAP
        return jnp.where(small, lo, mid), jnp.where(small, mid, hi)

    lo, hi = lax.fori_loop(0, 32, bs_body, (lo0, hi0))
    v = hi
    r = CAP - cnt_gt(v)

    tie = kmat == v

    def cnt_le(t):
        return jnp.sum((tie & (tok <= t)).astype(jnp.int32), axis=0, keepdims=True)

    lo0t = jnp.full((1, N_EXP), -1, jnp.int32)
    hi0t = jnp.full((1, N_EXP), N_TOK, jnp.int32)

    def ts_body(_, carry):
        lo, hi = carry
        mid = lo + (hi - lo) // 2
        ok = cnt_le(mid) <= r
        return jnp.where(ok, mid, lo), jnp.where(ok, hi, mid)

    lo, hi = lax.fori_loop(0, 13, ts_body, (lo0t, hi0t))
    tstar = lo

    mod_idx_cols = []
    mod_prob_cols = []
    keep0 = None
    sel0 = None
    for k in range(K_TOP):
        e_k = idx_ref[:, k : k + 1]
        p_k = prob_ref[:, k : k + 1]
        sel = lane == e_k
        v_k = jnp.sum(jnp.where(sel, v, 0), axis=1, keepdims=True)
        t_k = jnp.sum(jnp.where(sel, tstar, 0), axis=1, keepdims=True)
        key_k = lax.bitcast_convert_type(p_k, jnp.int32)
        keep = (key_k > v_k) | ((key_k == v_k) & (tokcol <= t_k))
        if k == 0:
            keep0 = keep
            sel0 = sel
        mod_prob_cols.append(jnp.where(keep, p_k, 0.0))
        mod_idx_cols.append(jnp.where(keep, e_k, -1))

    mod_idx_ref[...] = jnp.concatenate(mod_idx_cols, axis=1)
    mod_prob_ref[...] = jnp.concatenate(mod_prob_cols, axis=1)

    tpe = jnp.sum(
        jnp.where(sel0 & keep0, 1.0, 0.0).astype(jnp.float32), axis=0, keepdims=True
    )
    tpe_ref[...] = jnp.broadcast_to(tpe, tpe_ref.shape)

    colsum = colsum_ref[0:1, :]
    lb = jnp.sum(colsum * tpe) * (0.01 / (N_TOK * N_EXP))
    lb_ref[...] = jnp.broadcast_to(lb, lb_ref.shape)
    zl = (zacc_ref[0, 0] / N_TOK) * 0.001
    zl_ref[...] = jnp.broadcast_to(zl, zl_ref.shape)


@jax.jit
def kernel(x, W):
    idx, prob, keys, colsum, zacc = pl.pallas_call(
        _stage_a,
        grid=(N_BLK,),
        in_specs=[
            pl.BlockSpec((BLK, D_MODEL), lambda i: (i, 0)),
            pl.BlockSpec((N_EXP, D_MODEL), lambda i: (0, 0)),
        ],
        out_specs=[
            pl.BlockSpec((BLK, K_TOP), lambda i: (i, 0)),
            pl.BlockSpec((BLK, K_TOP), lambda i: (i, 0)),
            pl.BlockSpec((BLK, N_EXP), lambda i: (i, 0)),
            pl.BlockSpec((8, N_EXP), lambda i: (0, 0)),
            pl.BlockSpec((8, 128), lambda i: (0, 0)),
        ],
        out_shape=[
            jax.ShapeDtypeStruct((N_TOK, K_TOP), jnp.int32),
            jax.ShapeDtypeStruct((N_TOK, K_TOP), jnp.float32),
            jax.ShapeDtypeStruct((N_TOK, N_EXP), jnp.int32),
            jax.ShapeDtypeStruct((8, N_EXP), jnp.float32),
            jax.ShapeDtypeStruct((8, 128), jnp.float32),
        ],
    )(x, W)

    keysT = jnp.transpose(keys)
    vt = _sc_filter(keysT)
    v64 = jnp.broadcast_to(vt[:, 0:4:2].reshape(1, N_EXP), (8, N_EXP))
    t64 = jnp.broadcast_to(vt[:, 1:4:2].reshape(1, N_EXP), (8, N_EXP))

    mod_idx, mod_prob, tpe, lb, zl = pl.pallas_call(
        _stage_c,
        out_shape=[
            jax.ShapeDtypeStruct((N_TOK, K_TOP), jnp.int32),
            jax.ShapeDtypeStruct((N_TOK, K_TOP), jnp.float32),
            jax.ShapeDtypeStruct((8, N_EXP), jnp.float32),
            jax.ShapeDtypeStruct((8, 128), jnp.float32),
            jax.ShapeDtypeStruct((8, 128), jnp.float32),
        ],
    )(idx, prob, v64, t64, colsum, zacc)

    return (
        mod_idx,
        mod_prob,
        lb[0, 0],
        zl[0, 0],
        tpe[0, :],
    )


# speedup vs baseline: 1.8625x; 1.1300x over previous
import functools

import jax
import jax.numpy as jnp
from jax import lax
from jax.experimental import pallas as pl
from jax.experimental.pallas import tpu as pltpu
from jax.experimental.pallas import tpu_sc as plsc

D_MODEL = 4096
N_EXP = 64
K_TOP = 8
N_TOK = 8192
CAP = N_TOK // N_EXP
BLK = 512
N_BLK = N_TOK // BLK

_NEG_INF = float("-inf")


def _tree_sum8(vals):
    a = [vals[0] + vals[1], vals[2] + vals[3], vals[4] + vals[5], vals[6] + vals[7]]
    return (a[0] + a[1]) + (a[2] + a[3])


def _stage_a(x_ref, w_ref, idx_ref, prob_ref, keys_ref, colsum_ref, zacc_ref):
    pid = pl.program_id(0)
    x = x_ref[...]
    w = w_ref[...]
    logits = lax.dot_general(
        x, w, (((1,), (1,)), ((), ())), preferred_element_type=jnp.float32
    )

    lane = lax.broadcasted_iota(jnp.int32, (BLK, N_EXP), 1)

    m64 = jnp.max(logits, axis=1, keepdims=True)
    ex = jnp.exp(logits - m64)
    s64 = jnp.sum(ex, axis=1, keepdims=True)
    probs = ex / s64
    col_partial = jnp.sum(probs, axis=0, keepdims=True)
    lse = m64 + jnp.log(s64)
    z_partial = jnp.sum(lse * lse)

    l = logits
    vals = []
    idxs = []
    for _ in range(K_TOP):
        m = jnp.max(l, axis=1, keepdims=True)
        am = jnp.min(jnp.where(l == m, lane, N_EXP), axis=1, keepdims=True)
        vals.append(m)
        idxs.append(am)
        l = jnp.where(lane == am, _NEG_INF, l)

    exs = [jnp.exp(v - vals[0]) for v in vals]
    s8 = _tree_sum8(exs)
    ps = [e / s8 for e in exs]
    t8 = _tree_sum8(ps)
    t8 = jnp.maximum(t8, 1e-8)
    qs = [p / t8 for p in ps]

    keys = jnp.full((BLK, N_EXP), -1, jnp.int32)
    for k in range(K_TOP):
        kb = lax.bitcast_convert_type(qs[k], jnp.int32)
        keys = jnp.where(lane == idxs[k], kb, keys)

    idx_ref[...] = jnp.concatenate(idxs, axis=1)
    prob_ref[...] = jnp.concatenate(qs, axis=1)
    keys_ref[...] = keys

    @pl.when(pid == 0)
    def _():
        colsum_ref[...] = jnp.zeros_like(colsum_ref)
        zacc_ref[...] = jnp.zeros_like(zacc_ref)

    colsum_ref[...] += jnp.broadcast_to(col_partial, colsum_ref.shape)
    zacc_ref[...] += z_partial


_N_VREG = N_TOK // 16


_BSTRIDE = 144


def _sc_filter_body(keysT_ref, out_ref, keys_v, cand_k, cand_t, hist, res_v):
    c = lax.axis_index("c")
    s = lax.axis_index("s")
    wid = s * 2 + c
    lane16 = lax.iota(jnp.int32, 16)
    ones16 = jnp.ones((16,), jnp.int32)
    zeros16 = jnp.zeros((16,), jnp.int32)
    true16 = jnp.ones((16,), jnp.bool_)

    def vec_total(acc):
        t = acc[0]
        for l in range(1, 16):
            t = t + acc[l]
        return t

    res = jnp.zeros((16,), jnp.int32)
    for j in range(2):
        e = wid * 2 + j
        pltpu.sync_copy(keysT_ref.at[e], keys_v)

        def zbody(i, carry):
            hist[pl.ds(i * 16, 16)] = zeros16
            return carry

        lax.fori_loop(0, _BSTRIDE, zbody, 0, unroll=4)

        def hbody(i, carry):
            k = keys_v[pl.ds(i * 16, 16)]
            b = lax.shift_right_arithmetic(k, 23) + 1
            idx = lane16 * _BSTRIDE + b
            h = plsc.load_gather(hist, [idx])
            plsc.store_scatter(hist, [idx], h + 1)
            return carry

        lax.fori_loop(0, _N_VREG, hbody, 0, unroll=4)

        tot = [zeros16] * 9
        for l in range(16):
            for bv in range(9):
                tot[bv] = tot[bv] + hist[pl.ds(l * _BSTRIDE + bv * 16, 16)]

        tots = [vec_total(t) for t in tot]
        suffix = [None] * 9
        acc_s = jnp.int32(0)
        for v_i in range(8, -1, -1):
            suffix[v_i] = acc_s
            acc_s = acc_s + tots[v_i]
        vstar = jnp.int32(0)
        for v_i in range(9):
            cond = (suffix[v_i] < CAP) & (suffix[v_i] + tots[v_i] >= CAP)
            vstar = jnp.where(cond, v_i, vstar)
        tv = zeros16
        sfx = jnp.int32(0)
        for v_i in range(9):
            m_v = vstar == v_i
            tv = jnp.where(m_v, tot[v_i], tv)
            sfx = jnp.where(m_v, suffix[v_i], sfx)
        bstar = jnp.int32(0)
        c_above = jnp.int32(0)
        cum = sfx
        for l in range(15, -1, -1):
            tl = tv[l]
            cond = (cum < CAP) & (cum + tl >= CAP)
            bstar = jnp.where(cond, vstar * 16 + l, bstar)
            c_above = jnp.where(cond, cum, c_above)
            cum = cum + tl

        def cbody(i, cnt):
            k = keys_v[pl.ds(i * 16, 16)]
            b = lax.shift_right_arithmetic(k, 23) + 1
            m = b == bstar
            plsc.store_compressed(cand_k.at[pl.ds(cnt, 16)], k, mask=m)
            plsc.store_compressed(cand_t.at[pl.ds(cnt, 16)], lane16 + i * 16, mask=m)
            return cnt + plsc.all_reduce_population_count(m)[0]

        m_cnt = lax.fori_loop(0, _N_VREG, cbody, jnp.int32(0), unroll=2)
        nv_c = (m_cnt + 15) // 16

        lo0 = jnp.where(bstar == 0, jnp.int32(-2), ((bstar - 1) << 23) - 1)
        hi0 = jnp.where(bstar == 0, jnp.int32(0), bstar << 23)
        cap_r = CAP - c_above

        def cnt_gt(t):
            def body(i, acc):
                k = cand_k[pl.ds(i * 16, 16)]
                g = (lane16 + i * 16) < m_cnt
                return acc + jnp.where((k > t) & g, 1, 0).astype(jnp.int32)

            acc = lax.fori_loop(0, nv_c, body, zeros16)
            return vec_total(acc)

        def bs_body(_, lohi):
            lo, hi = lohi
            mid = lo + (hi - lo) // 2
            small = cnt_gt(mid) < cap_r
            return (jnp.where(small, lo, mid), jnp.where(small, mid, hi))

        lo, hi = lax.fori_loop(0, 24, bs_body, (lo0, hi0))
        v = hi
        r = cap_r - cnt_gt(v)

        def cnt_tle(t):
            def body(i, acc):
                k = cand_k[pl.ds(i * 16, 16)]
                ct = cand_t[pl.ds(i * 16, 16)]
                g = (lane16 + i * 16) < m_cnt
                m = (k == v) & (ct <= t) & g
                return acc + jnp.where(m, 1, 0).astype(jnp.int32)

            acc = lax.fori_loop(0, nv_c, body, zeros16)
            return vec_total(acc)

        def ts_body(_, lohi):
            lo, hi = lohi
            mid = lo + (hi - lo) // 2
            ok = cnt_tle(mid) <= r
            return (jnp.where(ok, mid, lo), jnp.where(ok, hi, mid))

        lo2, _ = lax.fori_loop(
            0, 13, ts_body, (jnp.int32(-1), jnp.int32(N_TOK))
        )

        res = jnp.where(lane16 == 2 * j, v, res)
        res = jnp.where(lane16 == 2 * j + 1, lo2, res)
    res_v[...] = res
    pltpu.sync_copy(res_v, out_ref.at[wid])


_sc_filter = functools.partial(
    pl.kernel,
    mesh=plsc.VectorSubcoreMesh(core_axis_name="c", subcore_axis_name="s"),
    out_type=jax.ShapeDtypeStruct((32, 16), jnp.int32),
    compiler_params=pltpu.CompilerParams(needs_layout_passes=False),
    scratch_types=[
        pltpu.VMEM((N_TOK,), jnp.int32),
        pltpu.VMEM((N_TOK + 16,), jnp.int32),
        pltpu.VMEM((N_TOK + 16,), jnp.int32),
        pltpu.VMEM((16 * _BSTRIDE,), jnp.int32),
        pltpu.VMEM((16,), jnp.int32),
    ],
)(_sc_filter_body)


def _stage_c(idx_ref, prob_ref, v_ref, t_ref, colsum_ref, zacc_ref,
             mod_idx_ref, mod_prob_ref, tpe_ref, lb_ref, zl_ref):
    lane = lax.broadcasted_iota(jnp.int32, (N_TOK, N_EXP), 1)
    tokcol = lax.broadcasted_iota(jnp.int32, (N_TOK, 1), 0)
    v = v_ref[0:1, :]
    tstar = t_ref[0:1, :]

    mod_idx_cols = []
    mod_prob_cols = []
    keep0 = None
    sel0 = None
    for k in range(K_TOP):
        e_k = idx_ref[:, k : k + 1]
        p_k = prob_ref[:, k : k + 1]
        sel = lane == e_k
        v_k = jnp.sum(jnp.where(sel, v, 0), axis=1, keepdims=True)
        t_k = jnp.sum(jnp.where(sel, tstar, 0), axis=1, keepdims=True)
        key_k = lax.bitcast_convert_type(p_k, jnp.int32)
        keep = (key_k > v_k) | ((key_k == v_k) & (tokcol <= t_k))
        if k == 0:
            keep0 = keep
            sel0 = sel
        mod_prob_cols.append(jnp.where(keep, p_k, 0.0))
        mod_idx_cols.append(jnp.where(keep, e_k, -1))

    mod_idx_ref[...] = jnp.concatenate(mod_idx_cols, axis=1)
    mod_prob_ref[...] = jnp.concatenate(mod_prob_cols, axis=1)

    tpe = jnp.sum(
        jnp.where(sel0 & keep0, 1.0, 0.0).astype(jnp.float32), axis=0, keepdims=True
    )
    tpe_ref[...] = jnp.broadcast_to(tpe, tpe_ref.shape)

    colsum = colsum_ref[0:1, :]
    lb = jnp.sum(colsum * tpe) * (0.01 / (N_TOK * N_EXP))
    lb_ref[...] = jnp.broadcast_to(lb, lb_ref.shape)
    zl = (zacc_ref[0, 0] / N_TOK) * 0.001
    zl_ref[...] = jnp.broadcast_to(zl, zl_ref.shape)


def _stage_b(idx_ref, prob_ref, keys_ref, colsum_ref, zacc_ref,
             mod_idx_ref, mod_prob_ref, tpe_ref, lb_ref, zl_ref):
    kmat = keys_ref[...]
    lane = lax.broadcasted_iota(jnp.int32, (N_TOK, N_EXP), 1)
    tok = lax.broadcasted_iota(jnp.int32, (N_TOK, N_EXP), 0)
    tokcol = lax.broadcasted_iota(jnp.int32, (N_TOK, 1), 0)

    def cnt_gt(t):
        return jnp.sum((kmat > t).astype(jnp.int32), axis=0, keepdims=True)

    lo0 = jnp.full((1, N_EXP), -2, jnp.int32)
    hi0 = jnp.full((1, N_EXP), 1 << 30, jnp.int32)

    def bs_body(_, carry):
        lo, hi = carry
        mid = lo + (hi - lo) // 2
        small = cnt_gt(mid) < CAP
        return jnp.where(small, lo, mid), jnp.where(small, mid, hi)

    lo, hi = lax.fori_loop(0, 32, bs_body, (lo0, hi0))
    v = hi
    r = CAP - cnt_gt(v)

    tie = kmat == v

    def cnt_le(t):
        return jnp.sum((tie & (tok <= t)).astype(jnp.int32), axis=0, keepdims=True)

    lo0t = jnp.full((1, N_EXP), -1, jnp.int32)
    hi0t = jnp.full((1, N_EXP), N_TOK, jnp.int32)

    def ts_body(_, carry):
        lo, hi = carry
        mid = lo + (hi - lo) // 2
        ok = cnt_le(mid) <= r
        return jnp.where(ok, mid, lo), jnp.where(ok, hi, mid)

    lo, hi = lax.fori_loop(0, 13, ts_body, (lo0t, hi0t))
    tstar = lo

    mod_idx_cols = []
    mod_prob_cols = []
    keep0 = None
    sel0 = None
    for k in range(K_TOP):
        e_k = idx_ref[:, k : k + 1]
        p_k = prob_ref[:, k : k + 1]
        sel = lane == e_k
        v_k = jnp.sum(jnp.where(sel, v, 0), axis=1, keepdims=True)
        t_k = jnp.sum(jnp.where(sel, tstar, 0), axis=1, keepdims=True)
        key_k = lax.bitcast_convert_type(p_k, jnp.int32)
        keep = (key_k > v_k) | ((key_k == v_k) & (tokcol <= t_k))
        if k == 0:
            keep0 = keep
            sel0 = sel
        mod_prob_cols.append(jnp.where(keep, p_k, 0.0))
        mod_idx_cols.append(jnp.where(keep, e_k, -1))

    mod_idx_ref[...] = jnp.concatenate(mod_idx_cols, axis=1)
    mod_prob_ref[...] = jnp.concatenate(mod_prob_cols, axis=1)

    tpe = jnp.sum(
        jnp.where(sel0 & keep0, 1.0, 0.0).astype(jnp.float32), axis=0, keepdims=True
    )
    tpe_ref[...] = jnp.broadcast_to(tpe, tpe_ref.shape)

    colsum = colsum_ref[0:1, :]
    lb = jnp.sum(colsum * tpe) * (0.01 / (N_TOK * N_EXP))
    lb_ref[...] = jnp.broadcast_to(lb, lb_ref.shape)
    zl = (zacc_ref[0, 0] / N_TOK) * 0.001
    zl_ref[...] = jnp.broadcast_to(zl, zl_ref.shape)


@jax.jit
def kernel(x, W):
    idx, prob, keys, colsum, zacc = pl.pallas_call(
        _stage_a,
        grid=(N_BLK,),
        in_specs=[
            pl.BlockSpec((BLK, D_MODEL), lambda i: (i, 0)),
            pl.BlockSpec((N_EXP, D_MODEL), lambda i: (0, 0)),
        ],
        out_specs=[
            pl.BlockSpec((BLK, K_TOP), lambda i: (i, 0)),
            pl.BlockSpec((BLK, K_TOP), lambda i: (i, 0)),
            pl.BlockSpec((BLK, N_EXP), lambda i: (i, 0)),
            pl.BlockSpec((8, N_EXP), lambda i: (0, 0)),
            pl.BlockSpec((8, 128), lambda i: (0, 0)),
        ],
        out_shape=[
            jax.ShapeDtypeStruct((N_TOK, K_TOP), jnp.int32),
            jax.ShapeDtypeStruct((N_TOK, K_TOP), jnp.float32),
            jax.ShapeDtypeStruct((N_TOK, N_EXP), jnp.int32),
            jax.ShapeDtypeStruct((8, N_EXP), jnp.float32),
            jax.ShapeDtypeStruct((8, 128), jnp.float32),
        ],
    )(x, W)

    keysT = jnp.transpose(keys)
    vt = _sc_filter(keysT)
    v64 = jnp.broadcast_to(vt[:, 0:4:2].reshape(1, N_EXP), (8, N_EXP))
    t64 = jnp.broadcast_to(vt[:, 1:4:2].reshape(1, N_EXP), (8, N_EXP))

    mod_idx, mod_prob, tpe, lb, zl = pl.pallas_call(
        _stage_c,
        out_shape=[
            jax.ShapeDtypeStruct((N_TOK, K_TOP), jnp.int32),
            jax.ShapeDtypeStruct((N_TOK, K_TOP), jnp.float32),
            jax.ShapeDtypeStruct((8, N_EXP), jnp.float32),
            jax.ShapeDtypeStruct((8, 128), jnp.float32),
            jax.ShapeDtypeStruct((8, 128), jnp.float32),
        ],
    )(idx, prob, v64, t64, colsum, zacc)

    return (
        mod_idx,
        mod_prob,
        lb[0, 0],
        zl[0, 0],
        tpe[0, :],
    )


# speedup vs baseline: 1.9211x; 1.0315x over previous
import functools

import jax
import jax.numpy as jnp
from jax import lax
from jax.experimental import pallas as pl
from jax.experimental.pallas import tpu as pltpu
from jax.experimental.pallas import tpu_sc as plsc

D_MODEL = 4096
N_EXP = 64
K_TOP = 8
N_TOK = 8192
CAP = N_TOK // N_EXP
BLK = 1024
N_BLK = N_TOK // BLK

_NEG_INF = float("-inf")


def _tree_sum8(vals):
    a = [vals[0] + vals[1], vals[2] + vals[3], vals[4] + vals[5], vals[6] + vals[7]]
    return (a[0] + a[1]) + (a[2] + a[3])


def _stage_a(x_ref, w_ref, idx_ref, prob_ref, keys_ref, colsum_ref, zacc_ref):
    pid = pl.program_id(0)
    x = x_ref[...]
    w = w_ref[...]
    logits = lax.dot_general(
        x, w, (((1,), (1,)), ((), ())), preferred_element_type=jnp.float32
    )

    lane = lax.broadcasted_iota(jnp.int32, (BLK, N_EXP), 1)

    m64 = jnp.max(logits, axis=1, keepdims=True)
    ex = jnp.exp(logits - m64)
    s64 = jnp.sum(ex, axis=1, keepdims=True)
    probs = ex / s64
    col_partial = jnp.sum(probs, axis=0, keepdims=True)
    lse = m64 + jnp.log(s64)
    z_partial = jnp.sum(lse * lse)

    l = logits
    vals = []
    idxs = []
    for _ in range(K_TOP):
        m = jnp.max(l, axis=1, keepdims=True)
        am = jnp.min(jnp.where(l == m, lane, N_EXP), axis=1, keepdims=True)
        vals.append(m)
        idxs.append(am)
        l = jnp.where(lane == am, _NEG_INF, l)

    exs = [jnp.exp(v - vals[0]) for v in vals]
    s8 = _tree_sum8(exs)
    ps = [e / s8 for e in exs]
    t8 = _tree_sum8(ps)
    t8 = jnp.maximum(t8, 1e-8)
    qs = [p / t8 for p in ps]

    keys = jnp.full((BLK, N_EXP), -1, jnp.int32)
    for k in range(K_TOP):
        kb = lax.bitcast_convert_type(qs[k], jnp.int32)
        keys = jnp.where(lane == idxs[k], kb, keys)

    idx_ref[...] = jnp.concatenate(idxs, axis=1)
    prob_ref[...] = jnp.concatenate(qs, axis=1)
    keys_ref[...] = keys

    @pl.when(pid == 0)
    def _():
        colsum_ref[...] = jnp.zeros_like(colsum_ref)
        zacc_ref[...] = jnp.zeros_like(zacc_ref)

    colsum_ref[...] += jnp.broadcast_to(col_partial, colsum_ref.shape)
    zacc_ref[...] += z_partial


_N_VREG = N_TOK // 16


_BSTRIDE = 144


def _sc_filter_body(keysT_ref, out_ref, keys_v, cand_k, cand_t, hist, res_v):
    c = lax.axis_index("c")
    s = lax.axis_index("s")
    wid = s * 2 + c
    lane16 = lax.iota(jnp.int32, 16)
    ones16 = jnp.ones((16,), jnp.int32)
    zeros16 = jnp.zeros((16,), jnp.int32)
    true16 = jnp.ones((16,), jnp.bool_)

    def vec_total(acc):
        t = acc[0]
        for l in range(1, 16):
            t = t + acc[l]
        return t

    res = jnp.zeros((16,), jnp.int32)
    for j in range(2):
        e = wid * 2 + j
        pltpu.sync_copy(keysT_ref.at[e], keys_v)

        def zbody(i, carry):
            hist[pl.ds(i * 16, 16)] = zeros16
            return carry

        lax.fori_loop(0, _BSTRIDE, zbody, 0, unroll=4)

        def hbody(i, carry):
            k = keys_v[pl.ds(i * 16, 16)]
            b = lax.shift_right_arithmetic(k, 23) + 1
            idx = lane16 * _BSTRIDE + b
            h = plsc.load_gather(hist, [idx])
            plsc.store_scatter(hist, [idx], h + 1)
            return carry

        lax.fori_loop(0, _N_VREG, hbody, 0, unroll=4)

        tot = [zeros16] * 9
        for l in range(16):
            for bv in range(9):
                tot[bv] = tot[bv] + hist[pl.ds(l * _BSTRIDE + bv * 16, 16)]

        tots = [vec_total(t) for t in tot]
        suffix = [None] * 9
        acc_s = jnp.int32(0)
        for v_i in range(8, -1, -1):
            suffix[v_i] = acc_s
            acc_s = acc_s + tots[v_i]
        vstar = jnp.int32(0)
        for v_i in range(9):
            cond = (suffix[v_i] < CAP) & (suffix[v_i] + tots[v_i] >= CAP)
            vstar = jnp.where(cond, v_i, vstar)
        tv = zeros16
        sfx = jnp.int32(0)
        for v_i in range(9):
            m_v = vstar == v_i
            tv = jnp.where(m_v, tot[v_i], tv)
            sfx = jnp.where(m_v, suffix[v_i], sfx)
        bstar = jnp.int32(0)
        c_above = jnp.int32(0)
        cum = sfx
        for l in range(15, -1, -1):
            tl = tv[l]
            cond = (cum < CAP) & (cum + tl >= CAP)
            bstar = jnp.where(cond, vstar * 16 + l, bstar)
            c_above = jnp.where(cond, cum, c_above)
            cum = cum + tl

        def cbody(i, cnt):
            k = keys_v[pl.ds(i * 16, 16)]
            b = lax.shift_right_arithmetic(k, 23) + 1
            m = b == bstar
            plsc.store_compressed(cand_k.at[pl.ds(cnt, 16)], k, mask=m)
            plsc.store_compressed(cand_t.at[pl.ds(cnt, 16)], lane16 + i * 16, mask=m)
            return cnt + plsc.all_reduce_population_count(m)[0]

        m_cnt = lax.fori_loop(0, _N_VREG, cbody, jnp.int32(0), unroll=2)
        nv_c = (m_cnt + 15) // 16

        lo0 = jnp.where(bstar == 0, jnp.int32(-2), ((bstar - 1) << 23) - 1)
        hi0 = jnp.where(bstar == 0, jnp.int32(0), bstar << 23)
        cap_r = CAP - c_above

        def cnt_gt(t):
            def body(i, acc):
                k = cand_k[pl.ds(i * 16, 16)]
                g = (lane16 + i * 16) < m_cnt
                return acc + jnp.where((k > t) & g, 1, 0).astype(jnp.int32)

            acc = lax.fori_loop(0, nv_c, body, zeros16)
            return vec_total(acc)

        def bs_body(_, lohi):
            lo, hi = lohi
            mid = lo + (hi - lo) // 2
            small = cnt_gt(mid) < cap_r
            return (jnp.where(small, lo, mid), jnp.where(small, mid, hi))

        lo, hi = lax.fori_loop(0, 24, bs_body, (lo0, hi0))
        v = hi
        r = cap_r - cnt_gt(v)

        def cnt_tle(t):
            def body(i, acc):
                k = cand_k[pl.ds(i * 16, 16)]
                ct = cand_t[pl.ds(i * 16, 16)]
                g = (lane16 + i * 16) < m_cnt
                m = (k == v) & (ct <= t) & g
                return acc + jnp.where(m, 1, 0).astype(jnp.int32)

            acc = lax.fori_loop(0, nv_c, body, zeros16)
            return vec_total(acc)

        def ts_body(_, lohi):
            lo, hi = lohi
            mid = lo + (hi - lo) // 2
            ok = cnt_tle(mid) <= r
            return (jnp.where(ok, mid, lo), jnp.where(ok, hi, mid))

        lo2, _ = lax.fori_loop(
            0, 13, ts_body, (jnp.int32(-1), jnp.int32(N_TOK))
        )

        res = jnp.where(lane16 == 2 * j, v, res)
        res = jnp.where(lane16 == 2 * j + 1, lo2, res)
    res_v[...] = res
    pltpu.sync_copy(res_v, out_ref.at[wid])


_sc_filter = functools.partial(
    pl.kernel,
    mesh=plsc.VectorSubcoreMesh(core_axis_name="c", subcore_axis_name="s"),
    out_type=jax.ShapeDtypeStruct((32, 16), jnp.int32),
    compiler_params=pltpu.CompilerParams(needs_layout_passes=False),
    scratch_types=[
        pltpu.VMEM((N_TOK,), jnp.int32),
        pltpu.VMEM((N_TOK + 16,), jnp.int32),
        pltpu.VMEM((N_TOK + 16,), jnp.int32),
        pltpu.VMEM((16 * _BSTRIDE,), jnp.int32),
        pltpu.VMEM((16,), jnp.int32),
    ],
)(_sc_filter_body)


def _stage_c(idx_ref, prob_ref, v_ref, t_ref, colsum_ref, zacc_ref,
             mod_idx_ref, mod_prob_ref, tpe_ref, lb_ref, zl_ref):
    lane = lax.broadcasted_iota(jnp.int32, (N_TOK, N_EXP), 1)
    tokcol = lax.broadcasted_iota(jnp.int32, (N_TOK, 1), 0)
    v = v_ref[0:1, :]
    tstar = t_ref[0:1, :]

    mod_idx_cols = []
    mod_prob_cols = []
    keep0 = None
    sel0 = None
    for k in range(K_TOP):
        e_k = idx_ref[:, k : k + 1]
        p_k = prob_ref[:, k : k + 1]
        sel = lane == e_k
        v_k = jnp.sum(jnp.where(sel, v, 0), axis=1, keepdims=True)
        t_k = jnp.sum(jnp.where(sel, tstar, 0), axis=1, keepdims=True)
        key_k = lax.bitcast_convert_type(p_k, jnp.int32)
        keep = (key_k > v_k) | ((key_k == v_k) & (tokcol <= t_k))
        if k == 0:
            keep0 = keep
            sel0 = sel
        mod_prob_cols.append(jnp.where(keep, p_k, 0.0))
        mod_idx_cols.append(jnp.where(keep, e_k, -1))

    mod_idx_ref[...] = jnp.concatenate(mod_idx_cols, axis=1)
    mod_prob_ref[...] = jnp.concatenate(mod_prob_cols, axis=1)

    tpe = jnp.sum(
        jnp.where(sel0 & keep0, 1.0, 0.0).astype(jnp.float32), axis=0, keepdims=True
    )
    tpe_ref[...] = jnp.broadcast_to(tpe, tpe_ref.shape)

    colsum = colsum_ref[0:1, :]
    lb = jnp.sum(colsum * tpe) * (0.01 / (N_TOK * N_EXP))
    lb_ref[...] = jnp.broadcast_to(lb, lb_ref.shape)
    zl = (zacc_ref[0, 0] / N_TOK) * 0.001
    zl_ref[...] = jnp.broadcast_to(zl, zl_ref.shape)


def _stage_b(idx_ref, prob_ref, keys_ref, colsum_ref, zacc_ref,
             mod_idx_ref, mod_prob_ref, tpe_ref, lb_ref, zl_ref):
    kmat = keys_ref[...]
    lane = lax.broadcasted_iota(jnp.int32, (N_TOK, N_EXP), 1)
    tok = lax.broadcasted_iota(jnp.int32, (N_TOK, N_EXP), 0)
    tokcol = lax.broadcasted_iota(jnp.int32, (N_TOK, 1), 0)

    def cnt_gt(t):
        return jnp.sum((kmat > t).astype(jnp.int32), axis=0, keepdims=True)

    lo0 = jnp.full((1, N_EXP), -2, jnp.int32)
    hi0 = jnp.full((1, N_EXP), 1 << 30, jnp.int32)

    def bs_body(_, carry):
        lo, hi = carry
        mid = lo + (hi - lo) // 2
        small = cnt_gt(mid) < CAP
        return jnp.where(small, lo, mid), jnp.where(small, mid, hi)

    lo, hi = lax.fori_loop(0, 32, bs_body, (lo0, hi0))
    v = hi
    r = CAP - cnt_gt(v)

    tie = kmat == v

    def cnt_le(t):
        return jnp.sum((tie & (tok <= t)).astype(jnp.int32), axis=0, keepdims=True)

    lo0t = jnp.full((1, N_EXP), -1, jnp.int32)
    hi0t = jnp.full((1, N_EXP), N_TOK, jnp.int32)

    def ts_body(_, carry):
        lo, hi = carry
        mid = lo + (hi - lo) // 2
        ok = cnt_le(mid) <= r
        return jnp.where(ok, mid, lo), jnp.where(ok, hi, mid)

    lo, hi = lax.fori_loop(0, 13, ts_body, (lo0t, hi0t))
    tstar = lo

    mod_idx_cols = []
    mod_prob_cols = []
    keep0 = None
    sel0 = None
    for k in range(K_TOP):
        e_k = idx_ref[:, k : k + 1]
        p_k = prob_ref[:, k : k + 1]
        sel = lane == e_k
        v_k = jnp.sum(jnp.where(sel, v, 0), axis=1, keepdims=True)
        t_k = jnp.sum(jnp.where(sel, tstar, 0), axis=1, keepdims=True)
        key_k = lax.bitcast_convert_type(p_k, jnp.int32)
        keep = (key_k > v_k) | ((key_k == v_k) & (tokcol <= t_k))
        if k == 0:
            keep0 = keep
            sel0 = sel
        mod_prob_cols.append(jnp.where(keep, p_k, 0.0))
        mod_idx_cols.append(jnp.where(keep, e_k, -1))

    mod_idx_ref[...] = jnp.concatenate(mod_idx_cols, axis=1)
    mod_prob_ref[...] = jnp.concatenate(mod_prob_cols, axis=1)

    tpe = jnp.sum(
        jnp.where(sel0 & keep0, 1.0, 0.0).astype(jnp.float32), axis=0, keepdims=True
    )
    tpe_ref[...] = jnp.broadcast_to(tpe, tpe_ref.shape)

    colsum = colsum_ref[0:1, :]
    lb = jnp.sum(colsum * tpe) * (0.01 / (N_TOK * N_EXP))
    lb_ref[...] = jnp.broadcast_to(lb, lb_ref.shape)
    zl = (zacc_ref[0, 0] / N_TOK) * 0.001
    zl_ref[...] = jnp.broadcast_to(zl, zl_ref.shape)


@jax.jit
def kernel(x, W):
    idx, prob, keys, colsum, zacc = pl.pallas_call(
        _stage_a,
        grid=(N_BLK,),
        in_specs=[
            pl.BlockSpec((BLK, D_MODEL), lambda i: (i, 0)),
            pl.BlockSpec((N_EXP, D_MODEL), lambda i: (0, 0)),
        ],
        out_specs=[
            pl.BlockSpec((BLK, K_TOP), lambda i: (i, 0)),
            pl.BlockSpec((BLK, K_TOP), lambda i: (i, 0)),
            pl.BlockSpec((BLK, N_EXP), lambda i: (i, 0)),
            pl.BlockSpec((8, N_EXP), lambda i: (0, 0)),
            pl.BlockSpec((8, 128), lambda i: (0, 0)),
        ],
        out_shape=[
            jax.ShapeDtypeStruct((N_TOK, K_TOP), jnp.int32),
            jax.ShapeDtypeStruct((N_TOK, K_TOP), jnp.float32),
            jax.ShapeDtypeStruct((N_TOK, N_EXP), jnp.int32),
            jax.ShapeDtypeStruct((8, N_EXP), jnp.float32),
            jax.ShapeDtypeStruct((8, 128), jnp.float32),
        ],
    )(x, W)

    keysT = jnp.transpose(keys)
    vt = _sc_filter(keysT)
    v64 = jnp.broadcast_to(vt[:, 0:4:2].reshape(1, N_EXP), (8, N_EXP))
    t64 = jnp.broadcast_to(vt[:, 1:4:2].reshape(1, N_EXP), (8, N_EXP))

    mod_idx, mod_prob, tpe, lb, zl = pl.pallas_call(
        _stage_c,
        out_shape=[
            jax.ShapeDtypeStruct((N_TOK, K_TOP), jnp.int32),
            jax.ShapeDtypeStruct((N_TOK, K_TOP), jnp.float32),
            jax.ShapeDtypeStruct((8, N_EXP), jnp.float32),
            jax.ShapeDtypeStruct((8, 128), jnp.float32),
            jax.ShapeDtypeStruct((8, 128), jnp.float32),
        ],
    )(idx, prob, v64, t64, colsum, zacc)

    return (
        mod_idx,
        mod_prob,
        lb[0, 0],
        zl[0, 0],
        tpe[0, :],
    )
